# baseline (speedup 1.0000x reference)
import functools

import jax
import jax.numpy as jnp
from jax import lax
from jax.experimental import pallas as pl
from jax.experimental.pallas import tpu as pltpu
from jax.experimental.pallas import tpu_sc as plsc

DIM = 32
NC = 2
NS = 16
NW = NC * NS

RPC = 16
HPAD = 128
H8 = 56
NSPLIT = 4


def _emb_kernel(batch: int, hist: int):
    rows_per_w = batch // NW
    n_chunks = rows_per_w // RPC
    assert n_chunks % 2 == 0
    mesh = plsc.VectorSubcoreMesh(
        core_axis_name="c", subcore_axis_name="s", num_cores=NC, num_subcores=NS
    )

    @functools.partial(
        pl.kernel,
        out_type=jax.ShapeDtypeStruct((batch, hist, DIM), jnp.float32),
        mesh=mesh,
        scratch_types=[
            pltpu.VMEM((rows_per_w, H8), jnp.int32),
            pltpu.VMEM((RPC, H8, DIM), jnp.float32),
            pltpu.VMEM((RPC, H8, DIM), jnp.float32),
            pltpu.SemaphoreType.DMA,
            pltpu.SemaphoreType.DMA,
        ],
        compiler_params=pltpu.CompilerParams(use_tc_tiling_on_sc=False),
    )
    def body(idx_hbm, table_hbm, out_hbm, idx_v, rows0, rows1, sem0, sem1):
        wid = lax.axis_index("s") * NC + lax.axis_index("c")
        base = wid * rows_per_w

        pltpu.sync_copy(
            idx_hbm.at[pl.ds(base, rows_per_w), pl.ds(0, H8)], idx_v
        )

        def gather(g, buf, sem):
            for r in range(RPC):
                pltpu.async_copy(
                    table_hbm.at[idx_v.at[g * RPC + r]], buf.at[r], sem
                )

        def drain(g, buf, sem):
            for r in range(RPC):
                pltpu.make_async_copy(
                    table_hbm.at[idx_v.at[g * RPC + r]], buf.at[r], sem
                ).wait()

        gather(0, rows0, sem0)

        bufs = (rows0, rows1)
        sems = (sem0, sem1)

        def outer(p, carry):
            for b in range(2):
                g = p * 2 + b
                cur, nxt = bufs[b], bufs[1 - b]
                csem, nsem = sems[b], sems[1 - b]

                @pl.when(g + 1 < n_chunks)
                def _():
                    gather(g + 1, nxt, nsem)

                drain(g, cur, csem)
                pltpu.sync_copy(
                    cur.at[:, pl.ds(0, hist), :],
                    out_hbm.at[pl.ds(base + g * RPC, RPC), :, :],
                )
            return carry

        lax.fori_loop(0, n_chunks // 2, outer, 0)

    return body


def kernel(input, weight):
    b, h = input.shape
    ipad = jnp.pad(input.astype(jnp.int32), ((0, 0), (0, HPAD - h)))
    part = _emb_kernel(b // NSPLIT, h)
    bs = b // NSPLIT
    parts = [part(ipad[i * bs : (i + 1) * bs], weight) for i in range(NSPLIT)]
    return jnp.concatenate(parts, axis=0)


# speedup vs baseline: 2.1107x; 2.1107x over previous
import functools

import jax
import jax.numpy as jnp
from jax import lax
from jax.experimental import pallas as pl
from jax.experimental.pallas import tpu as pltpu
from jax.experimental.pallas import tpu_sc as plsc

DIM = 32
NC = 2
NS = 16
NW = NC * NS

RPC = 16


def _emb_kernel(batch: int, hist: int):
    rows_per_w = batch // NW
    n_chunks = rows_per_w // RPC
    assert n_chunks % 2 == 0
    mesh = plsc.VectorSubcoreMesh(
        core_axis_name="c", subcore_axis_name="s", num_cores=NC, num_subcores=NS
    )

    @functools.partial(
        pl.kernel,
        out_type=jax.ShapeDtypeStruct((batch, hist, DIM), jnp.float32),
        mesh=mesh,
        scratch_types=[
            pltpu.VMEM((rows_per_w, hist), jnp.int32),
            pltpu.VMEM((RPC, hist, DIM), jnp.float32),
            pltpu.VMEM((RPC, hist, DIM), jnp.float32),
            pltpu.SemaphoreType.DMA,
            pltpu.SemaphoreType.DMA,
        ],
        compiler_params=pltpu.CompilerParams(use_tc_tiling_on_sc=False),
    )
    def body(idx_hbm, table_hbm, out_hbm, idx_v, rows0, rows1, sem0, sem1):
        wid = lax.axis_index("s") * NC + lax.axis_index("c")
        base = wid * rows_per_w

        pltpu.sync_copy(idx_hbm.at[pl.ds(base, rows_per_w), :], idx_v)

        def gather(g, buf, sem):
            for r in range(RPC):
                pltpu.async_copy(
                    table_hbm.at[idx_v.at[g * RPC + r]], buf.at[r], sem
                )

        def drain(g, buf, sem):
            for r in range(RPC):
                pltpu.make_async_copy(
                    table_hbm.at[idx_v.at[g * RPC + r]], buf.at[r], sem
                ).wait()

        gather(0, rows0, sem0)

        bufs = (rows0, rows1)
        sems = (sem0, sem1)

        def outer(p, carry):
            for b in range(2):
                g = p * 2 + b
                cur, nxt = bufs[b], bufs[1 - b]
                csem, nsem = sems[b], sems[1 - b]

                @pl.when(g + 1 < n_chunks)
                def _():
                    gather(g + 1, nxt, nsem)

                drain(g, cur, csem)
                pltpu.sync_copy(
                    cur, out_hbm.at[pl.ds(base + g * RPC, RPC), :, :]
                )
            return carry

        lax.fori_loop(0, n_chunks // 2, outer, 0)

    return body


def kernel(input, weight):
    b, h = input.shape
    idx = input.astype(jnp.int32)
    part = _emb_kernel(b // 4, h)
    bs = b // 4
    parts = [part(idx[i * bs : (i + 1) * bs], weight) for i in range(4)]
    return jnp.concatenate(parts, axis=0)
